# baseline (speedup 1.0000x reference)
import functools

import jax
import jax.numpy as jnp
from jax import lax
from jax.experimental import pallas as pl
from jax.experimental.pallas import tpu as pltpu
from jax.experimental.pallas import tpu_sc as plsc

_V = 26
_C = 16
_S = 100000
_B = 4096
_ROWS = _V * _C
_NC = 2
_NS = 16
_NW = _NC * _NS
_RPW = _ROWS // _NW
_CHUNK = 128
_NCHUNK = _B // _CHUNK
_LANES = 16
_TINY = 1.1754943508222875e-38


def _sc_gather_body(xt_hbm, wflat_hbm, g_hbm, xcol, idx, rowbuf, sem):
    wid = lax.axis_index("s") * _NC + lax.axis_index("c")

    def row_body(r, carry):
        row = wid * _RPW + r
        v = row // _C
        pltpu.sync_copy(xt_hbm.at[v], xcol)
        off = row * _S

        def build(j, carry):
            k = j // (_CHUNK // _LANES)
            l = j % (_CHUNK // _LANES)
            idx[k, pl.ds(l * _LANES, _LANES)] = (
                xcol[pl.ds(j * _LANES, _LANES)] + off
            )
            return carry

        lax.fori_loop(0, _B // _LANES, build, 0)

        def fire(k, carry):
            pltpu.async_copy(
                wflat_hbm.at[idx.at[k]],
                rowbuf.at[pl.ds(k * _CHUNK, _CHUNK)],
                sem,
            )
            return carry

        lax.fori_loop(0, _NCHUNK, fire, 0)

        def drain(k, carry):
            pltpu.make_async_copy(
                wflat_hbm.at[idx.at[k]],
                rowbuf.at[pl.ds(k * _CHUNK, _CHUNK)],
                sem,
            ).wait()
            return carry

        lax.fori_loop(0, _NCHUNK, drain, 0)

        pltpu.sync_copy(rowbuf, g_hbm.at[row])
        return carry

    lax.fori_loop(0, _RPW, row_body, 0)


_sc_gather = functools.partial(
    pl.kernel,
    out_type=jax.ShapeDtypeStruct((_ROWS, _B), jnp.float32),
    mesh=plsc.VectorSubcoreMesh(core_axis_name="c", subcore_axis_name="s"),
    scratch_types=[
        pltpu.VMEM((_B,), jnp.int32),
        pltpu.VMEM((_NCHUNK, _CHUNK), jnp.int32),
        pltpu.VMEM((_B,), jnp.float32),
        pltpu.SemaphoreType.DMA,
    ],
)(_sc_gather_body)


_BBLK = 512


def _tc_post_body(g_ref, w_ref, si_ref):
    g = g_ref[...]
    w = jnp.log(jnp.maximum(jnp.abs(g), _TINY))
    si = jnp.sign(g)
    w_ref[...] = w.T
    si_ref[...] = si.T


def _tc_post(g):
    return pl.pallas_call(
        _tc_post_body,
        grid=(_B // _BBLK,),
        in_specs=[pl.BlockSpec((_ROWS, _BBLK), lambda i: (0, i))],
        out_specs=[
            pl.BlockSpec((_BBLK, _ROWS), lambda i: (i, 0)),
            pl.BlockSpec((_BBLK, _ROWS), lambda i: (i, 0)),
        ],
        out_shape=[
            jax.ShapeDtypeStruct((_B, _ROWS), jnp.float32),
            jax.ShapeDtypeStruct((_B, _ROWS), jnp.float32),
        ],
    )(g)


def kernel(x, weight):
    xt = x.T
    wflat = weight.reshape(_ROWS * _S)
    g = _sc_gather(xt, wflat)
    w, si = _tc_post(g)
    w = w.reshape(_B, _V, 1, _C)
    si = si.reshape(_B, _V, 1, _C)
    return (w, si)


# speedup vs baseline: 2.2130x; 2.2130x over previous
import functools

import jax
import jax.numpy as jnp
from jax import lax
from jax.experimental import pallas as pl
from jax.experimental.pallas import tpu as pltpu
from jax.experimental.pallas import tpu_sc as plsc

_V = 26
_C = 16
_S = 100000
_B = 4096
_ROWS = _V * _C
_NC = 2
_NS = 16
_NW = _NC * _NS
_RPW = _ROWS // _NW
_LANES = 16
_TINY = 1.1754943508222875e-38
_LN2 = 0.6931471805599453

_P5 = 0.04392862784795105
_P4 = -0.409475585766429
_P3 = 1.610177546896671
_P2 = -3.520218838145311
_P1 = 5.0697563166331205
_P0 = -2.7941536765360095


def _sc_body(xt_hbm, w2d_hbm, w_hbm, si_hbm, xcol, rowbuf, wbuf, sibuf):
    wid = lax.axis_index("s") * _NC + lax.axis_index("c")

    def row_body(r, carry):
        row = wid * _RPW + r
        v = row // _C
        pltpu.sync_copy(xt_hbm.at[v], xcol)
        pltpu.sync_copy(w2d_hbm.at[row], rowbuf)

        def body(j, carry):
            sl = pl.ds(j * _LANES, _LANES)
            s = xcol[sl]
            g = plsc.load_gather(rowbuf, [s])
            one = jnp.float32(1.0)
            si = jnp.where(g > 0, one, jnp.float32(0.0)) - jnp.where(
                g < 0, one, jnp.float32(0.0)
            )
            a = jnp.maximum(jnp.abs(g), _TINY)
            bi = plsc.bitcast(a, jnp.int32)
            e = (bi >> 23) - 127
            m = plsc.bitcast(
                (bi & 0x007FFFFF) | 0x3F800000, jnp.float32
            )
            p = _P5 * m + _P4
            p = p * m + _P3
            p = p * m + _P2
            p = p * m + _P1
            p = p * m + _P0
            w = (e.astype(jnp.float32) + p) * _LN2
            wbuf[sl] = w
            sibuf[sl] = si
            return carry

        lax.fori_loop(0, _B // _LANES, body, 0)

        pltpu.sync_copy(wbuf, w_hbm.at[row])
        pltpu.sync_copy(sibuf, si_hbm.at[row])
        return carry

    lax.fori_loop(0, _RPW, row_body, 0)


_sc_lookup = functools.partial(
    pl.kernel,
    out_type=(
        jax.ShapeDtypeStruct((_ROWS, _B), jnp.float32),
        jax.ShapeDtypeStruct((_ROWS, _B), jnp.float32),
    ),
    mesh=plsc.VectorSubcoreMesh(core_axis_name="c", subcore_axis_name="s"),
    scratch_types=[
        pltpu.VMEM((_B,), jnp.int32),
        pltpu.VMEM((_S,), jnp.float32),
        pltpu.VMEM((_B,), jnp.float32),
        pltpu.VMEM((_B,), jnp.float32),
    ],
    compiler_params=pltpu.CompilerParams(needs_layout_passes=False),
)(_sc_body)


def kernel(x, weight):
    xt = x.T
    w2d = weight.reshape(_ROWS, _S)
    w, si = _sc_lookup(xt, w2d)
    w = w.T.reshape(_B, _V, 1, _C)
    si = si.T.reshape(_B, _V, 1, _C)
    return (w, si)
